# baseline (speedup 1.0000x reference)
import functools

import jax
import jax.numpy as jnp
from jax import lax
from jax.experimental import pallas as pl
from jax.experimental.pallas import tpu as pltpu
from jax.experimental.pallas import tpu_sc as plsc

N = 10000
E = 320000
D = 128

NC = 2
NS = 16
NW = NC * NS

CHUNK = 64
NCHUNK_P = 160
E_P = NW * NCHUNK_P * CHUNK
SEG = 80
NSEG = NCHUNK_P // SEG
N_ACC = 10112
ROWS_PT = N_ACC // NS
DUMMY = N

DEG_W = 8

RBLK = 2000
NBLK = N // RBLK

_mesh = plsc.VectorSubcoreMesh(core_axis_name="c", subcore_axis_name="s")


def _scatter_kernel_body(y_hbm, src_hbm, dst_hbm, zeros_hbm, part_hbm,
                         src_v, dst_v, rows0_v, rows1_v, acc_sh, sem0, sem1):
    c = lax.axis_index("c")
    s = lax.axis_index("s")
    w = c * NS + s
    pltpu.sync_copy(zeros_hbm.at[s], acc_sh.at[pl.ds(s * ROWS_PT, ROWS_PT)])
    plsc.subcore_barrier()

    def seg_loop(g, carry):
        pltpu.sync_copy(src_hbm.at[w].at[pl.ds(g * SEG, SEG)], src_v)
        pltpu.sync_copy(dst_hbm.at[w].at[pl.ds(g * SEG, SEG)], dst_v)

        def step(i, carry2):
            j = 2 * i
            cp0 = pltpu.async_copy(y_hbm.at[src_v.at[j]], rows0_v, sem0)
            cp1 = pltpu.async_copy(y_hbm.at[src_v.at[j + 1]], rows1_v, sem1)
            cp0.wait()
            pltpu.sync_copy(rows0_v, acc_sh.at[dst_v.at[j]], add=True)
            cp1.wait()
            pltpu.sync_copy(rows1_v, acc_sh.at[dst_v.at[j + 1]], add=True)
            return carry2

        lax.fori_loop(0, SEG // 2, step, 0)
        return carry

    lax.fori_loop(0, NSEG, seg_loop, 0)
    plsc.subcore_barrier()
    pltpu.sync_copy(acc_sh.at[pl.ds(s * ROWS_PT, ROWS_PT)], part_hbm.at[c].at[s])


_scatter_kernel = pl.kernel(
    _scatter_kernel_body,
    out_type=jax.ShapeDtypeStruct((NC, NS, ROWS_PT, D), jnp.float32),
    mesh=_mesh,
    scratch_types=[
        pltpu.VMEM((SEG, CHUNK), jnp.int32),
        pltpu.VMEM((SEG, CHUNK), jnp.int32),
        pltpu.VMEM((CHUNK, D), jnp.float32),
        pltpu.VMEM((CHUNK, D), jnp.float32),
        pltpu.VMEM_SHARED((N_ACC, D), jnp.float32),
        pltpu.SemaphoreType.DMA,
        pltpu.SemaphoreType.DMA,
    ],
)


def _dinv_from(degp_ref):
    deg = degp_ref[0, :, 0:1] + degp_ref[1, :, 0:1] + 1.0
    return lax.rsqrt(deg)


def _tc1_body(z_ref, w1_ref, degp_ref, y1_ref):
    dinv = _dinv_from(degp_ref)
    xw = jnp.dot(z_ref[...], w1_ref[...], preferred_element_type=jnp.float32)
    y1_ref[...] = xw * dinv


def _tc2_body(p_ref, y1_ref, degp_ref, b1_ref, w2_ref, y2_ref):
    dinv = _dinv_from(degp_ref)
    h = dinv * (p_ref[0] + p_ref[1] + y1_ref[...]) + b1_ref[...]
    h = jnp.where(h > 0, h, 0.01 * h)
    y2_ref[...] = jnp.dot(h, w2_ref[...], preferred_element_type=jnp.float32) * dinv


def _tc3_body(p_ref, y2_ref, degp_ref, b2_ref, out_ref):
    dinv = _dinv_from(degp_ref)
    out_ref[...] = dinv * (p_ref[0] + p_ref[1] + y2_ref[...]) + b2_ref[...]


_blk_nd = pl.BlockSpec((RBLK, D), lambda i: (i, 0))
_blk_pnd = pl.BlockSpec((NC, RBLK, D), lambda i: (0, i, 0))
_blk_deg = pl.BlockSpec((NC, RBLK, D), lambda i: (0, i, 0))
_blk_w = pl.BlockSpec((D, D), lambda i: (0, 0))
_blk_b = pl.BlockSpec((1, D), lambda i: (0, 0))

_tc1 = pl.pallas_call(
    _tc1_body,
    grid=(NBLK,),
    in_specs=[_blk_nd, _blk_w, _blk_deg],
    out_specs=_blk_nd,
    out_shape=jax.ShapeDtypeStruct((N, D), jnp.float32),
)
_tc2 = pl.pallas_call(
    _tc2_body,
    grid=(NBLK,),
    in_specs=[_blk_pnd, _blk_nd, _blk_deg, _blk_b, _blk_w],
    out_specs=_blk_nd,
    out_shape=jax.ShapeDtypeStruct((N, D), jnp.float32),
)
_tc3 = pl.pallas_call(
    _tc3_body,
    grid=(NBLK,),
    in_specs=[_blk_pnd, _blk_nd, _blk_deg, _blk_b],
    out_specs=_blk_nd,
    out_shape=jax.ShapeDtypeStruct((N, D), jnp.float32),
)


def kernel(z, edge_index, W1, b1, W2, b2):
    pad = E_P - E
    src3d = jnp.concatenate(
        [edge_index[0], jnp.zeros((pad,), jnp.int32)]).reshape(NW, NCHUNK_P, CHUNK)
    dst3d = jnp.concatenate(
        [edge_index[1], jnp.full((pad,), DUMMY, jnp.int32)]).reshape(NW, NCHUNK_P, CHUNK)
    zeros_nd = jnp.zeros((NS, ROWS_PT, D), jnp.float32)
    ones_nd = jnp.ones((N, D), jnp.float32)
    b1r = b1.reshape(1, D)
    b2r = b2.reshape(1, D)

    degp = _scatter_kernel(ones_nd, src3d, dst3d, zeros_nd).reshape(NC, N_ACC, D)
    y1 = _tc1(z, W1, degp)
    p1 = _scatter_kernel(y1, src3d, dst3d, zeros_nd).reshape(NC, N_ACC, D)
    y2 = _tc2(p1, y1, degp, b1r, W2)
    p2 = _scatter_kernel(y2, src3d, dst3d, zeros_nd).reshape(NC, N_ACC, D)
    return _tc3(p2, y2, degp, b2r)


# speedup vs baseline: 1.0008x; 1.0008x over previous
import functools

import jax
import jax.numpy as jnp
from jax import lax
from jax.experimental import pallas as pl
from jax.experimental.pallas import tpu as pltpu
from jax.experimental.pallas import tpu_sc as plsc

N = 10000
E = 320000
D = 128

NC = 2
NS = 16
NW = NC * NS

CHUNK = 64
NCHUNK_P = 160
E_P = NW * NCHUNK_P * CHUNK
SEG = 80
NSEG = NCHUNK_P // SEG
N_ACC = 10112
ROWS_PT = N_ACC // NS
DUMMY = N

DEG_W = 8

RBLK = 2000
NBLK = N // RBLK

_mesh = plsc.VectorSubcoreMesh(core_axis_name="c", subcore_axis_name="s")


def _scatter_kernel_body(y_hbm, src_hbm, dst_hbm, zeros_hbm, part_hbm,
                         src_v, dst_v, rows0_v, rows1_v, acc_sh, sem0, sem1):
    c = lax.axis_index("c")
    s = lax.axis_index("s")
    w = c * NS + s
    pltpu.sync_copy(zeros_hbm.at[s], acc_sh.at[pl.ds(s * ROWS_PT, ROWS_PT)])
    plsc.subcore_barrier()

    def seg_loop(g, carry):
        pltpu.sync_copy(src_hbm.at[w].at[pl.ds(g * SEG, SEG)], src_v)
        pltpu.sync_copy(dst_hbm.at[w].at[pl.ds(g * SEG, SEG)], dst_v)

        def step(i, carry2):
            j = 2 * i
            cp0 = pltpu.async_copy(y_hbm.at[src_v.at[j]], rows0_v, sem0)
            cp1 = pltpu.async_copy(y_hbm.at[src_v.at[j + 1]], rows1_v, sem1)
            cp0.wait()
            pltpu.sync_copy(rows0_v, acc_sh.at[dst_v.at[j]], add=True)
            cp1.wait()
            pltpu.sync_copy(rows1_v, acc_sh.at[dst_v.at[j + 1]], add=True)
            return carry2

        lax.fori_loop(0, SEG // 2, step, 0)
        return carry

    lax.fori_loop(0, NSEG, seg_loop, 0)
    plsc.subcore_barrier()
    pltpu.sync_copy(acc_sh.at[pl.ds(s * ROWS_PT, ROWS_PT)], part_hbm.at[c].at[s])


_scatter_kernel = pl.kernel(
    _scatter_kernel_body,
    out_type=jax.ShapeDtypeStruct((NC, NS, ROWS_PT, D), jnp.float32),
    mesh=_mesh,
    scratch_types=[
        pltpu.VMEM((SEG, CHUNK), jnp.int32),
        pltpu.VMEM((SEG, CHUNK), jnp.int32),
        pltpu.VMEM((CHUNK, D), jnp.float32),
        pltpu.VMEM((CHUNK, D), jnp.float32),
        pltpu.VMEM_SHARED((N_ACC, D), jnp.float32),
        pltpu.SemaphoreType.DMA,
        pltpu.SemaphoreType.DMA,
    ],
)


def _dinv_from(degp_ref):
    deg = degp_ref[0, :, 0:1] + degp_ref[1, :, 0:1] + 1.0
    return lax.rsqrt(deg)


def _tc1_body(z_ref, w1_ref, degp_ref, y1_ref):
    dinv = _dinv_from(degp_ref)
    xw = jnp.dot(z_ref[...], w1_ref[...], preferred_element_type=jnp.float32)
    y1_ref[...] = xw * dinv


def _tc2_body(p_ref, y1_ref, degp_ref, b1_ref, w2_ref, y2_ref):
    dinv = _dinv_from(degp_ref)
    h = dinv * (p_ref[0] + p_ref[1] + y1_ref[...]) + b1_ref[...]
    h = jnp.where(h > 0, h, 0.01 * h)
    y2_ref[...] = jnp.dot(h, w2_ref[...], preferred_element_type=jnp.float32) * dinv


def _tc3_body(p_ref, y2_ref, degp_ref, b2_ref, out_ref):
    dinv = _dinv_from(degp_ref)
    out_ref[...] = dinv * (p_ref[0] + p_ref[1] + y2_ref[...]) + b2_ref[...]


_blk_nd = pl.BlockSpec((RBLK, D), lambda i: (i, 0))
_blk_pnd = pl.BlockSpec((NC, RBLK, D), lambda i: (0, i, 0))
_blk_deg = pl.BlockSpec((NC, RBLK, D), lambda i: (0, i, 0))
_blk_w = pl.BlockSpec((D, D), lambda i: (0, 0))
_blk_b = pl.BlockSpec((1, D), lambda i: (0, 0))

_tc1 = pl.pallas_call(
    _tc1_body,
    grid=(NBLK,),
    in_specs=[_blk_nd, _blk_w, _blk_deg],
    out_specs=_blk_nd,
    out_shape=jax.ShapeDtypeStruct((N, D), jnp.float32),
)
_tc2 = pl.pallas_call(
    _tc2_body,
    grid=(NBLK,),
    in_specs=[_blk_pnd, _blk_nd, _blk_deg, _blk_b, _blk_w],
    out_specs=_blk_nd,
    out_shape=jax.ShapeDtypeStruct((N, D), jnp.float32),
)
_tc3 = pl.pallas_call(
    _tc3_body,
    grid=(NBLK,),
    in_specs=[_blk_pnd, _blk_nd, _blk_deg, _blk_b],
    out_specs=_blk_nd,
    out_shape=jax.ShapeDtypeStruct((N, D), jnp.float32),
)


def kernel(z, edge_index, W1, b1, W2, b2):
    pad = E_P - E
    src3d = jnp.concatenate(
        [edge_index[0], jnp.zeros((pad,), jnp.int32)]).reshape(NW, NCHUNK_P, CHUNK)
    dummy_rows = DUMMY + (jnp.arange(pad, dtype=jnp.int32) % (N_ACC - N))
    dst3d = jnp.concatenate(
        [edge_index[1], dummy_rows]).reshape(NW, NCHUNK_P, CHUNK)
    zeros_nd = jnp.zeros((NS, ROWS_PT, D), jnp.float32)
    ones_nd = jnp.ones((N, D), jnp.float32)
    b1r = b1.reshape(1, D)
    b2r = b2.reshape(1, D)

    degp = _scatter_kernel(ones_nd, src3d, dst3d, zeros_nd).reshape(NC, N_ACC, D)
    y1 = _tc1(z, W1, degp)
    p1 = _scatter_kernel(y1, src3d, dst3d, zeros_nd).reshape(NC, N_ACC, D)
    y2 = _tc2(p1, y1, degp, b1r, W2)
    p2 = _scatter_kernel(y2, src3d, dst3d, zeros_nd).reshape(NC, N_ACC, D)
    return _tc3(p2, y2, degp, b2r)


# speedup vs baseline: 2.3234x; 2.3215x over previous
import functools

import jax
import jax.numpy as jnp
from jax import lax
from jax.experimental import pallas as pl
from jax.experimental.pallas import tpu as pltpu
from jax.experimental.pallas import tpu_sc as plsc

N = 10000
E = 320000
D = 128

NC = 2
NS = 16
NW = NC * NS

CHUNK = 80
NCHUNK_P = 125
N_ACC = N
ROWS_PT = N_ACC // NS

DEG_W = 8

RBLK = 2000
NBLK = N // RBLK

_mesh = plsc.VectorSubcoreMesh(core_axis_name="c", subcore_axis_name="s")


def _scatter_kernel_body(y_hbm, src_hbm, dst_hbm, zeros_hbm, part_hbm,
                         src_v, dst_v, rows_v, acc_sh, sem):
    c = lax.axis_index("c")
    s = lax.axis_index("s")
    w = c * NS + s
    pltpu.sync_copy(zeros_hbm.at[s], acc_sh.at[pl.ds(s * ROWS_PT, ROWS_PT)])
    pltpu.sync_copy(src_hbm.at[w], src_v)
    pltpu.sync_copy(dst_hbm.at[w], dst_v)
    plsc.subcore_barrier()

    def step(j, carry):
        pltpu.async_copy(y_hbm.at[src_v.at[j]], rows_v, sem).wait()
        pltpu.sync_copy(rows_v, acc_sh.at[dst_v.at[j]], add=True)
        return carry

    lax.fori_loop(0, NCHUNK_P, step, 0)
    plsc.subcore_barrier()
    pltpu.sync_copy(acc_sh.at[pl.ds(s * ROWS_PT, ROWS_PT)], part_hbm.at[c].at[s])


_scatter_kernel = pl.kernel(
    _scatter_kernel_body,
    out_type=jax.ShapeDtypeStruct((NC, NS, ROWS_PT, D), jnp.float32),
    mesh=_mesh,
    scratch_types=[
        pltpu.VMEM((NCHUNK_P, CHUNK), jnp.int32),
        pltpu.VMEM((NCHUNK_P, CHUNK), jnp.int32),
        pltpu.VMEM((CHUNK, D), jnp.float32),
        pltpu.VMEM_SHARED((N_ACC, D), jnp.float32),
        pltpu.SemaphoreType.DMA,
    ],
)


def _dinv_from(degp_ref):
    deg = degp_ref[0, :, 0:1] + degp_ref[1, :, 0:1] + 1.0
    return lax.rsqrt(deg)


def _tc1_body(z_ref, w1_ref, degp_ref, y1_ref):
    dinv = _dinv_from(degp_ref)
    xw = jnp.dot(z_ref[...], w1_ref[...], preferred_element_type=jnp.float32)
    y1_ref[...] = xw * dinv


def _tc2_body(p_ref, y1_ref, degp_ref, b1_ref, w2_ref, y2_ref):
    dinv = _dinv_from(degp_ref)
    h = dinv * (p_ref[0] + p_ref[1] + y1_ref[...]) + b1_ref[...]
    h = jnp.where(h > 0, h, 0.01 * h)
    y2_ref[...] = jnp.dot(h, w2_ref[...], preferred_element_type=jnp.float32) * dinv


def _tc3_body(p_ref, y2_ref, degp_ref, b2_ref, out_ref):
    dinv = _dinv_from(degp_ref)
    out_ref[...] = dinv * (p_ref[0] + p_ref[1] + y2_ref[...]) + b2_ref[...]


_blk_nd = pl.BlockSpec((RBLK, D), lambda i: (i, 0))
_blk_pnd = pl.BlockSpec((NC, RBLK, D), lambda i: (0, i, 0))
_blk_deg = pl.BlockSpec((NC, RBLK, D), lambda i: (0, i, 0))
_blk_w = pl.BlockSpec((D, D), lambda i: (0, 0))
_blk_b = pl.BlockSpec((1, D), lambda i: (0, 0))

_tc1 = pl.pallas_call(
    _tc1_body,
    grid=(NBLK,),
    in_specs=[_blk_nd, _blk_w, _blk_deg],
    out_specs=_blk_nd,
    out_shape=jax.ShapeDtypeStruct((N, D), jnp.float32),
)
_tc2 = pl.pallas_call(
    _tc2_body,
    grid=(NBLK,),
    in_specs=[_blk_pnd, _blk_nd, _blk_deg, _blk_b, _blk_w],
    out_specs=_blk_nd,
    out_shape=jax.ShapeDtypeStruct((N, D), jnp.float32),
)
_tc3 = pl.pallas_call(
    _tc3_body,
    grid=(NBLK,),
    in_specs=[_blk_pnd, _blk_nd, _blk_deg, _blk_b],
    out_specs=_blk_nd,
    out_shape=jax.ShapeDtypeStruct((N, D), jnp.float32),
)


def kernel(z, edge_index, W1, b1, W2, b2):
    src3d = edge_index[0].reshape(NW, NCHUNK_P, CHUNK)
    dst3d = edge_index[1].reshape(NW, NCHUNK_P, CHUNK)
    zeros_nd = jnp.zeros((NS, ROWS_PT, D), jnp.float32)
    ones_nd = jnp.ones((N, D), jnp.float32)
    b1r = b1.reshape(1, D)
    b2r = b2.reshape(1, D)

    degp = _scatter_kernel(ones_nd, src3d, dst3d, zeros_nd).reshape(NC, N_ACC, D)
    y1 = _tc1(z, W1, degp)
    p1 = _scatter_kernel(y1, src3d, dst3d, zeros_nd).reshape(NC, N_ACC, D)
    y2 = _tc2(p1, y1, degp, b1r, W2)
    p2 = _scatter_kernel(y2, src3d, dst3d, zeros_nd).reshape(NC, N_ACC, D)
    return _tc3(p2, y2, degp, b2r)
